# baseline (speedup 1.0000x reference)
import math
from functools import partial

import numpy as np
import jax
import jax.numpy as jnp
from jax.experimental import pallas as pl
from jax.experimental.pallas import tpu as pltpu


def _bilinear_matrix(in_size: int, out_size: int) -> np.ndarray:
    scale = in_size / out_size
    src = (np.arange(out_size, dtype=np.float64) + 0.5) * scale - 0.5
    src = np.maximum(src, 0.0)
    i0 = np.minimum(np.floor(src).astype(np.int64), in_size - 1)
    i1 = np.minimum(i0 + 1, in_size - 1)
    frac = src - i0
    m = np.zeros((out_size, in_size), dtype=np.float64)
    rows = np.arange(out_size)
    m[rows, i0] += 1.0 - frac
    m[rows, i1] += frac
    return m


def _staged_bilinear(size: int, multiplier: float, n_stages: int) -> np.ndarray:
    m = np.eye(size, dtype=np.float64)
    cur = size
    for _ in range(n_stages):
        nxt = int(math.floor(cur * multiplier))
        m = _bilinear_matrix(cur, nxt) @ m
        cur = nxt
    return m


def _pool2_matrix(size: int) -> np.ndarray:
    return np.kron(np.eye(size // 2), np.array([[0.5, 0.5]]))


def _iota_pool_awt(W, Wo):
    r = jax.lax.broadcasted_iota(jnp.int32, (W, Wo), 0)
    c = jax.lax.broadcasted_iota(jnp.int32, (W, Wo), 1)
    return jnp.where(r // 2 == c, 0.5, 0.0).astype(jnp.bfloat16)


def _iota_pool_ah(Ho, H):
    r = jax.lax.broadcasted_iota(jnp.int32, (Ho, H), 0)
    c = jax.lax.broadcasted_iota(jnp.int32, (Ho, H), 1)
    return jnp.where(c // 2 == r, 0.5, 0.0).astype(jnp.bfloat16)


def _rescale_body(x_ref, w_ref, b_ref, *rest, BB, NS, C, H, Ho, Wo, Cout, pool2):
    W = x_ref.shape[2]
    if pool2:
        o_hbm, o_buf, out_sem = rest
        awt = _iota_pool_awt(W, Wo)
        ah = _iota_pool_ah(Ho, H)
    else:
        awt_ref, ah_ref, o_hbm, o_buf, out_sem = rest
        awt = awt_ref[...]
        ah = ah_ref[...]
    NSLOT = o_buf.shape[0]
    i = pl.program_id(0)
    slot = jax.lax.rem(i, NSLOT)
    o_ref = o_buf.at[slot]
    x = x_ref[...].reshape(BB * C * H, W).astype(jnp.bfloat16)
    y = jnp.dot(x, awt, preferred_element_type=jnp.float32)
    yh = y.astype(jnp.bfloat16)
    for b in range(BB):
        z = [jnp.dot(ah, yh[(b * C + c) * H:(b * C + c + 1) * H, :],
                     preferred_element_type=jnp.float32)
             for c in range(C)]
        for co in range(Cout):
            acc = z[0] * w_ref[co, 0]
            for c in range(1, C):
                acc = acc + z[c] * w_ref[co, c]
            o_ref[b, co * Ho:(co + 1) * Ho, :] = acc + b_ref[co]
    pltpu.make_async_copy(o_buf.at[slot], o_hbm.at[pl.ds(i * BB, BB)],
                          out_sem.at[slot]).start()

    @pl.when(jax.lax.rem(i, NSLOT) == NSLOT - 1)
    def _drain():
        for j in range(NSLOT):
            pltpu.make_async_copy(o_buf.at[j], o_hbm.at[pl.ds(0, BB)],
                                  out_sem.at[j]).wait()


def kernel(x, w_map, b_map):
    N, C, H, W = x.shape
    Cout = int(w_map.shape[0])
    a_h = _staged_bilinear(H, 0.5, 1)
    a_w = _staged_bilinear(W, 0.5, 1)
    Ho, Wo = a_h.shape[0], a_w.shape[0]

    pool2 = (H % 2 == 0 and W % 2 == 0
             and np.array_equal(a_h, _pool2_matrix(H))
             and np.array_equal(a_w, _pool2_matrix(W)))

    BB = next((b for b in (4, 2) if N % (2 * b * 4) == 0), None)
    NSLOT = 4
    if BB is None:
        BB = next((b for b in (8, 4, 2) if N % b == 0 and N // b >= 2), 1)
        NSLOT = min(2, N // BB)
    NS = N // BB
    x_in = x.reshape(N, C * H, W)

    in_specs = [
        pl.BlockSpec((BB, C * H, W), lambda n: (n, 0, 0)),
        pl.BlockSpec(memory_space=pltpu.SMEM),
        pl.BlockSpec(memory_space=pltpu.SMEM),
    ]
    inputs = [x_in, jnp.asarray(w_map, jnp.float32), jnp.asarray(b_map, jnp.float32)]
    if not pool2:
        in_specs += [pl.BlockSpec((W, Wo), lambda n: (0, 0)),
                     pl.BlockSpec((Ho, H), lambda n: (0, 0))]
        inputs += [jnp.asarray(a_w.T.astype(np.float32)).astype(jnp.bfloat16),
                   jnp.asarray(a_h.astype(np.float32)).astype(jnp.bfloat16)]

    out = pl.pallas_call(
        partial(_rescale_body, BB=BB, NS=NS, C=C, H=H, Ho=Ho, Wo=Wo,
                Cout=Cout, pool2=pool2),
        out_shape=jax.ShapeDtypeStruct((N, Cout * Ho, Wo), x.dtype),
        grid=(NS,),
        in_specs=in_specs,
        out_specs=pl.BlockSpec(memory_space=pl.ANY),
        scratch_shapes=[
            pltpu.VMEM((NSLOT, BB, Cout * Ho, Wo), x.dtype),
            pltpu.SemaphoreType.DMA((NSLOT,)),
        ],
        compiler_params=pltpu.CompilerParams(
            dimension_semantics=("parallel",),
            vmem_limit_bytes=100 * 1024 * 1024,
        ),
    )(*inputs)
    return out.reshape(N, Cout, Ho, Wo)


# speedup vs baseline: 1.0916x; 1.0916x over previous
import math
from functools import partial

import numpy as np
import jax
import jax.numpy as jnp
from jax.experimental import pallas as pl
from jax.experimental.pallas import tpu as pltpu


def _bilinear_matrix(in_size: int, out_size: int) -> np.ndarray:
    scale = in_size / out_size
    src = (np.arange(out_size, dtype=np.float64) + 0.5) * scale - 0.5
    src = np.maximum(src, 0.0)
    i0 = np.minimum(np.floor(src).astype(np.int64), in_size - 1)
    i1 = np.minimum(i0 + 1, in_size - 1)
    frac = src - i0
    m = np.zeros((out_size, in_size), dtype=np.float64)
    rows = np.arange(out_size)
    m[rows, i0] += 1.0 - frac
    m[rows, i1] += frac
    return m


def _staged_bilinear(size: int, multiplier: float, n_stages: int) -> np.ndarray:
    m = np.eye(size, dtype=np.float64)
    cur = size
    for _ in range(n_stages):
        nxt = int(math.floor(cur * multiplier))
        m = _bilinear_matrix(cur, nxt) @ m
        cur = nxt
    return m


def _pool2_matrix(size: int) -> np.ndarray:
    return np.kron(np.eye(size // 2), np.array([[0.5, 0.5]]))


def _iota_pool_awt(W, Wo):
    r = jax.lax.broadcasted_iota(jnp.int32, (W, Wo), 0)
    c = jax.lax.broadcasted_iota(jnp.int32, (W, Wo), 1)
    return jnp.where(r // 2 == c, 0.5, 0.0).astype(jnp.float32)


def _iota_pool_ah(Ho, H):
    r = jax.lax.broadcasted_iota(jnp.int32, (Ho, H), 0)
    c = jax.lax.broadcasted_iota(jnp.int32, (Ho, H), 1)
    return jnp.where(c // 2 == r, 0.5, 0.0).astype(jnp.float32)


def _rescale_body(x_ref, w_ref, b_ref, *rest, BB, C, H, Ho, Wo, Cout, pool2):
    W = x_ref.shape[2]
    if pool2:
        (o_ref,) = rest
        awt = _iota_pool_awt(W, Wo)
        ah = _iota_pool_ah(Ho, H)
    else:
        awt_ref, ah_ref, o_ref = rest
        awt = awt_ref[...]
        ah = ah_ref[...]
    x = x_ref[...].reshape(BB * C * H, W)
    y = jnp.dot(x, awt, preferred_element_type=jnp.float32)
    yh = y
    for b in range(BB):
        z = [jnp.dot(ah, yh[(b * C + c) * H:(b * C + c + 1) * H, :],
                     preferred_element_type=jnp.float32)
             for c in range(C)]
        for co in range(Cout):
            acc = z[0] * w_ref[co, 0]
            for c in range(1, C):
                acc = acc + z[c] * w_ref[co, c]
            o_ref[b, co * Ho:(co + 1) * Ho, :] = acc + b_ref[co]


def kernel(x, w_map, b_map):
    N, C, H, W = x.shape
    Cout = int(w_map.shape[0])
    a_h = _staged_bilinear(H, 0.5, 1)
    a_w = _staged_bilinear(W, 0.5, 1)
    Ho, Wo = a_h.shape[0], a_w.shape[0]

    pool2 = (H % 2 == 0 and W % 2 == 0
             and np.array_equal(a_h, _pool2_matrix(H))
             and np.array_equal(a_w, _pool2_matrix(W)))

    BB = next((b for b in (8, 4, 2) if N % b == 0 and N // b >= 2), 1)
    x_in = x.reshape(N, C * H, W)

    in_specs = [
        pl.BlockSpec((BB, C * H, W), lambda n: (n, 0, 0)),
        pl.BlockSpec(memory_space=pltpu.SMEM),
        pl.BlockSpec(memory_space=pltpu.SMEM),
    ]
    inputs = [x_in, jnp.asarray(w_map, jnp.float32), jnp.asarray(b_map, jnp.float32)]
    if not pool2:
        in_specs += [pl.BlockSpec((W, Wo), lambda n: (0, 0)),
                     pl.BlockSpec((Ho, H), lambda n: (0, 0))]
        inputs += [jnp.asarray(a_w.T.astype(np.float32)).astype(jnp.bfloat16),
                   jnp.asarray(a_h.astype(np.float32)).astype(jnp.bfloat16)]

    out = pl.pallas_call(
        partial(_rescale_body, BB=BB, C=C, H=H, Ho=Ho, Wo=Wo, Cout=Cout,
                pool2=pool2),
        out_shape=jax.ShapeDtypeStruct((N, Cout * Ho, Wo), x.dtype),
        grid=(N // BB,),
        in_specs=in_specs,
        out_specs=pl.BlockSpec((BB, Cout * Ho, Wo), lambda n: (n, 0, 0)),
        compiler_params=pltpu.CompilerParams(
            dimension_semantics=("parallel",),
            vmem_limit_bytes=100 * 1024 * 1024,
        ),
    )(*inputs)
    return out.reshape(N, Cout, Ho, Wo)
